# Initial kernel scaffold; baseline (speedup 1.0000x reference)
import functools

import jax
import jax.numpy as jnp
from jax import lax
from jax.experimental import pallas as pl
from jax.experimental.pallas import tpu as pltpu
from jax.experimental.pallas import tpu_sc as plsc

U_N, I_N, D, K, LAYERS = 50000, 50000, 64, 3, 2
DH = D // 2
NC, NS = 2, 16
CB = 1024
G = CB // 128
NOUT = 50000
NPAD = 50048
ZR = 782
HN = 100000
HPAD = 100096
E_UU, E_II, E_UI = 800000, 800000, 600000

_mesh = plsc.VectorSubcoreMesh(
    core_axis_name="c", subcore_axis_name="s", num_cores=NC, num_subcores=NS)


def _zero_fill(zbuf, nrows, ncol16):
    zvec = jnp.zeros((16,), jnp.float32)

    def zb(i, _):
        for j in range(ncol16):
            zbuf[i, pl.ds(j * 16, 16)] = zvec
        return 0

    lax.fori_loop(0, nrows, zb, 0)


def _make_segsum(nchunks):
    ept128 = nchunks * G

    @functools.partial(
        pl.kernel,
        mesh=_mesh,
        out_type=(jax.ShapeDtypeStruct((NOUT, DH), jnp.float32),
                  jax.ShapeDtypeStruct((NOUT, DH), jnp.float32)),
        scratch_types=(
            pltpu.VMEM_SHARED((NPAD, DH), jnp.float32),
            pltpu.VMEM((G, 128), jnp.int32),
            pltpu.VMEM((G, 128), jnp.int32),
            pltpu.VMEM((CB, DH), jnp.float32),
            pltpu.VMEM((ZR, DH), jnp.float32),
            pltpu.SemaphoreType.DMA,
        ),
    )
    def segsum(xl, xr, src2, dst2, outl, outr, acc, sidx, didx, rows, zbuf,
               gsem):
        c = lax.axis_index("c")
        s = lax.axis_index("s")
        _zero_fill(zbuf, ZR, DH // 16)
        for z in range(NPAD // NS // ZR):
            pltpu.sync_copy(zbuf, acc.at[pl.ds(s * (NPAD // NS) + z * ZR, ZR)])
        plsc.subcore_barrier()

        base128 = s * ept128

        def chunk(g, _):
            off = base128 + g * G
            pltpu.sync_copy(src2.at[pl.ds(off, G)], sidx)
            pltpu.sync_copy(dst2.at[pl.ds(off, G)], didx)

            @pl.when(c == 0)
            def _():
                ds = [pltpu.async_copy(xl.at[sidx.at[j]],
                                       rows.at[pl.ds(j * 128, 128)], gsem)
                      for j in range(G)]
                for dsc in ds:
                    dsc.wait()

            @pl.when(c == 1)
            def _():
                ds = [pltpu.async_copy(xr.at[sidx.at[j]],
                                       rows.at[pl.ds(j * 128, 128)], gsem)
                      for j in range(G)]
                for dsc in ds:
                    dsc.wait()

            for j in range(G):
                pltpu.sync_copy(rows.at[pl.ds(j * 128, 128)],
                                acc.at[didx.at[j]], add=True)
            return 0

        lax.fori_loop(0, nchunks, chunk, 0)
        plsc.subcore_barrier()

        rpt = NOUT // NS

        @pl.when(c == 0)
        def _():
            pltpu.sync_copy(acc.at[pl.ds(s * rpt, rpt)],
                            outl.at[pl.ds(s * rpt, rpt)])

        @pl.when(c == 1)
        def _():
            pltpu.sync_copy(acc.at[pl.ds(s * rpt, rpt)],
                            outr.at[pl.ds(s * rpt, rpt)])

    return segsum


_SEGSUM_CACHE = {}


def _segsum(xs, src, dst):
    E = src.shape[0]
    epad = -(-E // (NS * CB)) * (NS * CB)
    nch = epad // (NS * CB)
    if nch not in _SEGSUM_CACHE:
        _SEGSUM_CACHE[nch] = _make_segsum(nch)
    pad_s = jnp.full((epad - E,), 0, jnp.int32)
    pad_d = jnp.full((epad - E,), NOUT, jnp.int32)
    s2 = jnp.concatenate([src, pad_s]).reshape(-1, 128)
    d2 = jnp.concatenate([dst, pad_d]).reshape(-1, 128)
    ol, orr = _SEGSUM_CACHE[nch](xs[:, :DH], xs[:, DH:], s2, d2)
    return jnp.concatenate([ol, orr], axis=1)


_H0 = E_UU + E_II
_H1 = 2 * E_UI
_H0P = -(-_H0 // (NS * CB)) * (NS * CB)
_H1P = -(-_H1 // (NS * CB)) * (NS * CB)
_H0CH = _H0P // (NS * CB)
_H1CH = _H1P // (NS * CB)


@functools.partial(
    pl.kernel,
    mesh=_mesh,
    out_type=jax.ShapeDtypeStruct((2 * HN, 16), jnp.float32),
    scratch_types=(
        pltpu.VMEM_SHARED((HPAD, 16), jnp.float32),
        pltpu.VMEM((G, 128), jnp.int32),
        pltpu.VMEM((CB, 16), jnp.float32),
        pltpu.VMEM((ZR, 16), jnp.float32),
    ),
)
def _hist_kernel(dst2, out, acc, didx, ones, zbuf):
    c = lax.axis_index("c")
    s = lax.axis_index("s")
    _zero_fill(zbuf, ZR, 1)
    for z in range(HPAD // NS // ZR):
        pltpu.sync_copy(zbuf, acc.at[pl.ds(s * (HPAD // NS) + z * ZR, ZR)])

    ovec = jnp.ones((16,), jnp.float32)

    def ob(i, _):
        ones[i, pl.ds(0, 16)] = ovec
        return 0

    lax.fori_loop(0, CB, ob, 0)
    plsc.subcore_barrier()

    ept128_0 = _H0CH * G
    ept128_1 = _H1CH * G
    nch = jnp.where(c == 0, _H0CH, _H1CH)
    base128 = jnp.where(c == 0, s * ept128_0, _H0P // 128 + s * ept128_1)

    def chunk(g, _):
        off = base128 + g * G
        pltpu.sync_copy(dst2.at[pl.ds(off, G)], didx)
        for j in range(G):
            pltpu.sync_copy(ones.at[pl.ds(j * 128, 128)],
                            acc.at[didx.at[j]], add=True)
        return 0

    lax.fori_loop(0, nch, chunk, 0)
    plsc.subcore_barrier()

    rpt = HN // NS
    pltpu.sync_copy(acc.at[pl.ds(s * rpt, rpt)],
                    out.at[pl.ds(c * HN + s * rpt, rpt)])


def _degrees(uu_row, ii_row, ui_row, ui_col):
    p0 = jnp.full((_H0P - _H0,), HN, jnp.int32)
    p1 = jnp.full((_H1P - _H1,), HN, jnp.int32)
    d = jnp.concatenate([
        uu_row, ii_row + U_N, p0,
        ui_row, ui_col + U_N, p1,
    ]).reshape(-1, 128)
    cnt = _hist_kernel(d)[:, 0]
    return cnt[:U_N], cnt[U_N:2 * U_N], cnt[2 * U_N:3 * U_N], cnt[3 * U_N:]


def _l2norm(x):
    n = jnp.sqrt(jnp.sum(x * x, axis=-1, keepdims=True))
    return x / jnp.maximum(n, 1e-12)


def _dis(cnt):
    return jnp.where(cnt > 0, lax.rsqrt(jnp.maximum(cnt, 1e-12)), 0.0)


def _mlp(x, pw, pb, a, ow, ob):
    h = x @ pw + pb
    h = jnp.where(h >= 0, h, a * h)
    h = h @ ow + ob
    return _l2norm(h)


def kernel(user_emb, item_emb, gating_weightu, gating_weightub,
           gating_weighti, gating_weightib, meta_netu_w, meta_netu_b,
           meta_neti_w, meta_neti_b, mlp_pre_w, mlp_pre_b, mlp_prelu,
           mlp_out_w, mlp_out_b, uu_row, uu_col, ii_row, ii_col, ui_row,
           ui_col):
    cnt_uu, cnt_ii, cnt_su, cnt_si = _degrees(uu_row, ii_row, ui_row, ui_col)
    dis_uu = _dis(cnt_uu)[:, None]
    dis_ii = _dis(cnt_ii)[:, None]
    dis_su = _dis(cnt_su)[:, None]
    dis_si = _dis(cnt_si)[:, None]

    uu0 = user_emb * jax.nn.sigmoid(user_emb @ gating_weightu + gating_weightub)
    ii0 = item_emb * jax.nn.sigmoid(item_emb @ gating_weighti + gating_weightib)

    all_u, all_i = [uu0], [ii0]
    all_ui_u, all_ui_i = [user_emb], [item_emb]
    uemb, iemb = uu0, ii0
    ui_u, ui_i = user_emb, item_emb
    for _ in range(LAYERS):
        u0 = dis_uu * _segsum(dis_uu * uemb, uu_col, uu_row)
        i0 = dis_ii * _segsum(dis_ii * iemb, ii_col, ii_row)
        ui_lu = dis_su * _segsum(dis_si * ui_i, ui_col, ui_row)
        ui_li = dis_si * _segsum(dis_su * ui_u, ui_row, ui_col)
        uemb = (u0 + ui_lu) / 2.0
        iemb = (i0 + ui_li) / 2.0
        ui_u, ui_i = uemb, iemb
        all_u.append(_l2norm(u0))
        all_i.append(_l2norm(i0))
        all_ui_u.append(_l2norm(ui_lu))
        all_ui_i.append(_l2norm(ui_li))

    userEmbedding = sum(all_u) / (LAYERS + 1)
    itemEmbedding = sum(all_i) / (LAYERS + 1)
    ui_userEmbedding = sum(all_ui_u) / (LAYERS + 1)
    ui_itemEmbedding = sum(all_ui_i) / (LAYERS + 1)

    uneighbor = _segsum(ui_itemEmbedding, ui_col, ui_row)
    ineighbor = _segsum(ui_userEmbedding, ui_row, ui_col)
    tembedu = jnp.concatenate([userEmbedding, ui_userEmbedding, uneighbor],
                              1) @ meta_netu_w + meta_netu_b
    tembedi = jnp.concatenate([itemEmbedding, ui_itemEmbedding, ineighbor],
                              1) @ meta_neti_w + meta_neti_b
    metau1 = _mlp(tembedu, mlp_pre_w[0], mlp_pre_b[0], mlp_prelu[0],
                  mlp_out_w[0], mlp_out_b[0]).reshape(-1, D, K)
    metau2 = _mlp(tembedu, mlp_pre_w[1], mlp_pre_b[1], mlp_prelu[1],
                  mlp_out_w[1], mlp_out_b[1]).reshape(-1, K, D)
    metai1 = _mlp(tembedi, mlp_pre_w[2], mlp_pre_b[2], mlp_prelu[2],
                  mlp_out_w[2], mlp_out_b[2]).reshape(-1, D, K)
    metai2 = _mlp(tembedi, mlp_pre_w[3], mlp_pre_b[3], mlp_prelu[3],
                  mlp_out_w[3], mlp_out_b[3]).reshape(-1, K, D)
    lw_u1 = jax.nn.softmax(metau1 + jnp.mean(metau1, 0), axis=1)
    lw_u2 = jax.nn.softmax(metau2 + jnp.mean(metau2, 0), axis=1)
    lw_i1 = jax.nn.softmax(metai1 + jnp.mean(metai1, 0), axis=1)
    lw_i2 = jax.nn.softmax(metai2 + jnp.mean(metai2, 0), axis=1)
    tu = jnp.sum(userEmbedding[:, :, None] * lw_u1, axis=1)
    tu = jnp.sum(tu[:, :, None] * lw_u2, axis=1)
    ti = jnp.sum(itemEmbedding[:, :, None] * lw_i1, axis=1)
    ti = jnp.sum(ti[:, :, None] * lw_i2, axis=1)
    userEmbedding = userEmbedding + tu
    itemEmbedding = itemEmbedding + ti
    return userEmbedding, itemEmbedding, ui_userEmbedding, ui_itemEmbedding


# speedup vs baseline: 8.1892x; 8.1892x over previous
import functools

import jax
import jax.numpy as jnp
from jax import lax
from jax.experimental import pallas as pl
from jax.experimental.pallas import tpu as pltpu
from jax.experimental.pallas import tpu_sc as plsc

U_N, I_N, D, K, LAYERS = 50000, 50000, 64, 3, 2
DH = D // 2
NC, NS = 2, 16
CB = 512
G = CB // 128
NOUT = 50000
NPAD = 50176
ZR = 224
HN = 100000
HPAD = 100352
E_UU, E_II, E_UI = 800000, 800000, 600000

_MESH_CACHE = []


def _mesh():
    if not _MESH_CACHE:
        _MESH_CACHE.append(plsc.VectorSubcoreMesh(
            core_axis_name="c", subcore_axis_name="s",
            num_cores=NC, num_subcores=NS))
    return _MESH_CACHE[0]


def _zero_fill(zbuf, nrows, ncol16):
    zvec = jnp.zeros((16,), jnp.float32)

    def zb(i, _):
        for j in range(ncol16):
            zbuf[i, pl.ds(j * 16, 16)] = zvec
        return 0

    lax.fori_loop(0, nrows, zb, 0)


def _make_segsum(nchunks):
    ept128 = nchunks * G

    @functools.partial(
        pl.kernel,
        mesh=_mesh(),
        out_type=jax.ShapeDtypeStruct((NPAD, NC, DH), jnp.float32),
        scratch_types=(
            pltpu.VMEM_SHARED((NPAD, DH), jnp.float32),
            pltpu.VMEM((G, 128), jnp.int32),
            pltpu.VMEM((G, 128), jnp.int32),
            pltpu.VMEM((CB, DH), jnp.float32),
            pltpu.VMEM((ZR, DH), jnp.float32),
            pltpu.SemaphoreType.DMA,
        ),
        compiler_params=pltpu.CompilerParams(use_tc_tiling_on_sc=False),
    )
    def segsum(x2, src2, dst2, out, acc, sidx, didx, rows, zbuf, gsem):
        c = lax.axis_index("c")
        s = lax.axis_index("s")
        _zero_fill(zbuf, ZR, DH // 16)
        for z in range(NPAD // NS // ZR):
            pltpu.sync_copy(zbuf, acc.at[pl.ds(s * (NPAD // NS) + z * ZR, ZR)])
        plsc.subcore_barrier()

        base128 = s * ept128

        def chunk(g, _):
            off = base128 + g * G
            pltpu.sync_copy(src2.at[pl.ds(off, G)], sidx)
            pltpu.sync_copy(dst2.at[pl.ds(off, G)], didx)

            for j in range(G):
                for h in range(8):
                    v = sidx[j, pl.ds(h * 16, 16)]
                    sidx[j, pl.ds(h * 16, 16)] = v + v + c
            ds = [pltpu.async_copy(x2.at[sidx.at[j]],
                                   rows.at[pl.ds(j * 128, 128)], gsem)
                  for j in range(G)]
            for dsc in ds:
                dsc.wait()

            for j in range(G):
                pltpu.sync_copy(rows.at[pl.ds(j * 128, 128)],
                                acc.at[didx.at[j]], add=True)
            return 0

        lax.fori_loop(0, nchunks, chunk, 0)
        plsc.subcore_barrier()

        rpt = NPAD // NS
        pltpu.sync_copy(acc.at[pl.ds(s * rpt, rpt)],
                        out.at[pl.ds(s * rpt, rpt), c])

    return segsum


_SEGSUM_CACHE = {}


def _segsum(xs, src, dst):
    E = src.shape[0]
    epad = -(-E // (NS * CB)) * (NS * CB)
    nch = epad // (NS * CB)
    if nch not in _SEGSUM_CACHE:
        _SEGSUM_CACHE[nch] = _make_segsum(nch)
    pad_s = jnp.full((epad - E,), 0, jnp.int32)
    pad_d = jnp.full((epad - E,), NOUT, jnp.int32)
    s2 = jnp.concatenate([src, pad_s]).reshape(-1, 128)
    d2 = jnp.concatenate([dst, pad_d]).reshape(-1, 128)
    x2 = xs.reshape(-1, DH)
    out = _SEGSUM_CACHE[nch](x2, s2, d2)
    return out[:NOUT].reshape(NOUT, D)


_H0 = E_UU + E_II
_H1 = 2 * E_UI
_H0P = -(-_H0 // (NS * CB)) * (NS * CB)
_H1P = -(-_H1 // (NS * CB)) * (NS * CB)
_H0CH = _H0P // (NS * CB)
_H1CH = _H1P // (NS * CB)


_HIST_CACHE = []


def _make_hist():
    @functools.partial(
        pl.kernel,
        mesh=_mesh(),
        out_type=jax.ShapeDtypeStruct((2 * HPAD, 16), jnp.float32),
        scratch_types=(
            pltpu.VMEM_SHARED((HPAD, 16), jnp.float32),
            pltpu.VMEM((G, 128), jnp.int32),
            pltpu.VMEM((CB, 16), jnp.float32),
            pltpu.VMEM((ZR, 16), jnp.float32),
        ),
        compiler_params=pltpu.CompilerParams(use_tc_tiling_on_sc=False),
    )
    def _hist_kernel(dst2, out, acc, didx, ones, zbuf):
        c = lax.axis_index("c")
        s = lax.axis_index("s")
        _zero_fill(zbuf, ZR, 1)
        for z in range(HPAD // NS // ZR):
            pltpu.sync_copy(zbuf, acc.at[pl.ds(s * (HPAD // NS) + z * ZR, ZR)])

        ovec = jnp.ones((16,), jnp.float32)

        def ob(i, _):
            ones[i, pl.ds(0, 16)] = ovec
            return 0

        lax.fori_loop(0, CB, ob, 0)
        plsc.subcore_barrier()

        ept128_0 = _H0CH * G
        ept128_1 = _H1CH * G
        nch = jnp.where(c == 0, _H0CH, _H1CH)
        base128 = jnp.where(c == 0, s * ept128_0, _H0P // 128 + s * ept128_1)

        def chunk(g, _):
            off = base128 + g * G
            pltpu.sync_copy(dst2.at[pl.ds(off, G)], didx)
            for j in range(G):
                pltpu.sync_copy(ones.at[pl.ds(j * 128, 128)],
                                acc.at[didx.at[j]], add=True)
            return 0

        lax.fori_loop(0, nch, chunk, 0)
        plsc.subcore_barrier()

        rpt = HPAD // NS
        pltpu.sync_copy(acc.at[pl.ds(s * rpt, rpt)],
                        out.at[pl.ds(c * HPAD + s * rpt, rpt)])

    return _hist_kernel


def _hist_call(d):
    if not _HIST_CACHE:
        _HIST_CACHE.append(_make_hist())
    return _HIST_CACHE[0](d)


def _degrees(uu_row, ii_row, ui_row, ui_col):
    p0 = jnp.full((_H0P - _H0,), HN, jnp.int32)
    p1 = jnp.full((_H1P - _H1,), HN, jnp.int32)
    d = jnp.concatenate([
        uu_row, ii_row + U_N, p0,
        ui_row, ui_col + U_N, p1,
    ]).reshape(-1, 128)
    cnt = _hist_call(d)[:, 0]
    return (cnt[:U_N], cnt[U_N:2 * U_N],
            cnt[HPAD:HPAD + U_N], cnt[HPAD + U_N:HPAD + 2 * U_N])


def _l2norm(x):
    n = jnp.sqrt(jnp.sum(x * x, axis=-1, keepdims=True))
    return x / jnp.maximum(n, 1e-12)


def _dis(cnt):
    return jnp.where(cnt > 0, lax.rsqrt(jnp.maximum(cnt, 1e-12)), 0.0)


def _mlp(x, pw, pb, a, ow, ob):
    h = x @ pw + pb
    h = jnp.where(h >= 0, h, a * h)
    h = h @ ow + ob
    return _l2norm(h)


def kernel(user_emb, item_emb, gating_weightu, gating_weightub,
           gating_weighti, gating_weightib, meta_netu_w, meta_netu_b,
           meta_neti_w, meta_neti_b, mlp_pre_w, mlp_pre_b, mlp_prelu,
           mlp_out_w, mlp_out_b, uu_row, uu_col, ii_row, ii_col, ui_row,
           ui_col):
    cnt_uu, cnt_ii, cnt_su, cnt_si = _degrees(uu_row, ii_row, ui_row, ui_col)
    dis_uu = _dis(cnt_uu)[:, None]
    dis_ii = _dis(cnt_ii)[:, None]
    dis_su = _dis(cnt_su)[:, None]
    dis_si = _dis(cnt_si)[:, None]

    uu0 = user_emb * jax.nn.sigmoid(user_emb @ gating_weightu + gating_weightub)
    ii0 = item_emb * jax.nn.sigmoid(item_emb @ gating_weighti + gating_weightib)

    all_u, all_i = [uu0], [ii0]
    all_ui_u, all_ui_i = [user_emb], [item_emb]
    uemb, iemb = uu0, ii0
    ui_u, ui_i = user_emb, item_emb
    for _ in range(LAYERS):
        u0 = dis_uu * _segsum(dis_uu * uemb, uu_col, uu_row)
        i0 = dis_ii * _segsum(dis_ii * iemb, ii_col, ii_row)
        ui_lu = dis_su * _segsum(dis_si * ui_i, ui_col, ui_row)
        ui_li = dis_si * _segsum(dis_su * ui_u, ui_row, ui_col)
        uemb = (u0 + ui_lu) / 2.0
        iemb = (i0 + ui_li) / 2.0
        ui_u, ui_i = uemb, iemb
        all_u.append(_l2norm(u0))
        all_i.append(_l2norm(i0))
        all_ui_u.append(_l2norm(ui_lu))
        all_ui_i.append(_l2norm(ui_li))

    userEmbedding = sum(all_u) / (LAYERS + 1)
    itemEmbedding = sum(all_i) / (LAYERS + 1)
    ui_userEmbedding = sum(all_ui_u) / (LAYERS + 1)
    ui_itemEmbedding = sum(all_ui_i) / (LAYERS + 1)

    uneighbor = _segsum(ui_itemEmbedding, ui_col, ui_row)
    ineighbor = _segsum(ui_userEmbedding, ui_row, ui_col)
    tembedu = jnp.concatenate([userEmbedding, ui_userEmbedding, uneighbor],
                              1) @ meta_netu_w + meta_netu_b
    tembedi = jnp.concatenate([itemEmbedding, ui_itemEmbedding, ineighbor],
                              1) @ meta_neti_w + meta_neti_b
    metau1 = _mlp(tembedu, mlp_pre_w[0], mlp_pre_b[0], mlp_prelu[0],
                  mlp_out_w[0], mlp_out_b[0]).reshape(-1, D, K)
    metau2 = _mlp(tembedu, mlp_pre_w[1], mlp_pre_b[1], mlp_prelu[1],
                  mlp_out_w[1], mlp_out_b[1]).reshape(-1, K, D)
    metai1 = _mlp(tembedi, mlp_pre_w[2], mlp_pre_b[2], mlp_prelu[2],
                  mlp_out_w[2], mlp_out_b[2]).reshape(-1, D, K)
    metai2 = _mlp(tembedi, mlp_pre_w[3], mlp_pre_b[3], mlp_prelu[3],
                  mlp_out_w[3], mlp_out_b[3]).reshape(-1, K, D)
    lw_u1 = jax.nn.softmax(metau1 + jnp.mean(metau1, 0), axis=1)
    lw_u2 = jax.nn.softmax(metau2 + jnp.mean(metau2, 0), axis=1)
    lw_i1 = jax.nn.softmax(metai1 + jnp.mean(metai1, 0), axis=1)
    lw_i2 = jax.nn.softmax(metai2 + jnp.mean(metai2, 0), axis=1)
    tu = jnp.sum(userEmbedding[:, :, None] * lw_u1, axis=1)
    tu = jnp.sum(tu[:, :, None] * lw_u2, axis=1)
    ti = jnp.sum(itemEmbedding[:, :, None] * lw_i1, axis=1)
    ti = jnp.sum(ti[:, :, None] * lw_i2, axis=1)
    userEmbedding = userEmbedding + tu
    itemEmbedding = itemEmbedding + ti
    return userEmbedding, itemEmbedding, ui_userEmbedding, ui_itemEmbedding


# speedup vs baseline: 8.6114x; 1.0516x over previous
import functools

import jax
import jax.numpy as jnp
from jax import lax
from jax.experimental import pallas as pl
from jax.experimental.pallas import tpu as pltpu
from jax.experimental.pallas import tpu_sc as plsc

U_N, I_N, D, K, LAYERS = 50000, 50000, 64, 3, 2
DH = D // 2
NC, NS = 2, 16
CB = 384
G = CB // 128
NOUT = 50000
NPAD = 50176
ZR = 112
HN = 100000
HPAD = 100352
E_UU, E_II, E_UI = 800000, 800000, 600000

_MESH_CACHE = []


def _mesh():
    if not _MESH_CACHE:
        _MESH_CACHE.append(plsc.VectorSubcoreMesh(
            core_axis_name="c", subcore_axis_name="s",
            num_cores=NC, num_subcores=NS))
    return _MESH_CACHE[0]


def _zero_fill(zbuf, nrows, ncol16):
    zvec = jnp.zeros((16,), jnp.float32)

    def zb(i, _):
        for j in range(ncol16):
            zbuf[i, pl.ds(j * 16, 16)] = zvec
        return 0

    lax.fori_loop(0, nrows, zb, 0)


def _make_segsum(nchunks):
    ept128 = nchunks * G

    @functools.partial(
        pl.kernel,
        mesh=_mesh(),
        out_type=jax.ShapeDtypeStruct((NPAD, NC, DH), jnp.float32),
        scratch_types=(
            pltpu.VMEM_SHARED((NPAD, DH), jnp.float32),
            pltpu.VMEM((G, 128), jnp.int32),
            pltpu.VMEM((G, 128), jnp.int32),
            pltpu.VMEM((CB, DH), jnp.float32),
            pltpu.VMEM((G, 128), jnp.int32),
            pltpu.VMEM((G, 128), jnp.int32),
            pltpu.VMEM((CB, DH), jnp.float32),
            pltpu.VMEM((ZR, DH), jnp.float32),
            pltpu.SemaphoreType.DMA,
            pltpu.SemaphoreType.DMA,
            pltpu.SemaphoreType.DMA,
            pltpu.SemaphoreType.DMA,
            pltpu.SemaphoreType.DMA,
            pltpu.SemaphoreType.DMA,
        ),
        compiler_params=pltpu.CompilerParams(use_tc_tiling_on_sc=False),
    )
    def segsum(x2, src2, dst2, out, acc, sidx0, didx0, rows0, sidx1, didx1,
               rows1, zbuf, isem0, isem1, gsem0, gsem1, ssem0, ssem1):
        c = lax.axis_index("c")
        s = lax.axis_index("s")
        _zero_fill(zbuf, ZR, DH // 16)
        for z in range(NPAD // NS // ZR):
            pltpu.sync_copy(zbuf, acc.at[pl.ds(s * (NPAD // NS) + z * ZR, ZR)])
        plsc.subcore_barrier()

        base128 = s * ept128
        cohorts = ((sidx0, didx0, rows0, isem0, gsem0, ssem0),
                   (sidx1, didx1, rows1, isem1, gsem1, ssem1))

        def fire_idx(i, coh):
            off = base128 + jnp.minimum(i, nchunks - 1) * G
            pltpu.async_copy(src2.at[pl.ds(off, G)], coh[0], coh[3])
            pltpu.async_copy(dst2.at[pl.ds(off, G)], coh[1], coh[3])

        fire_idx(0, cohorts[0])

        def pair(i0, _):
            for b in range(2):
                sidx, didx, rows, isem, gsem, ssem = cohorts[b]
                ocoh = cohorts[1 - b]
                i = 2 * i0 + b
                pltpu.make_async_copy(src2.at[pl.ds(base128, G)], sidx,
                                      isem).wait()
                pltpu.make_async_copy(dst2.at[pl.ds(base128, G)], didx,
                                      isem).wait()
                for j in range(G):
                    for h in range(8):
                        v = sidx[j, pl.ds(h * 16, 16)]
                        sidx[j, pl.ds(h * 16, 16)] = v + v + c
                for j in range(G):
                    pltpu.async_copy(x2.at[sidx.at[j]],
                                     rows.at[pl.ds(j * 128, 128)], gsem)

                @pl.when(i >= 1)
                def _():
                    for j in range(G):
                        pltpu.make_async_copy(
                            ocoh[2].at[pl.ds(j * 128, 128)],
                            acc.at[ocoh[1].at[j]], ocoh[5]).wait()

                fire_idx(i + 1, ocoh)
                for j in range(G):
                    pltpu.make_async_copy(x2.at[sidx.at[j]],
                                          rows.at[pl.ds(j * 128, 128)],
                                          gsem).wait()
                for j in range(G):
                    pltpu.async_copy(rows.at[pl.ds(j * 128, 128)],
                                     acc.at[didx.at[j]], ssem, add=True)
            return 0

        lax.fori_loop(0, nchunks // 2, pair, 0)
        lastc = cohorts[1]
        for j in range(G):
            pltpu.make_async_copy(lastc[2].at[pl.ds(j * 128, 128)],
                                  acc.at[lastc[1].at[j]], lastc[5]).wait()
        coh = cohorts[0]
        pltpu.make_async_copy(src2.at[pl.ds(base128, G)], coh[0],
                              coh[3]).wait()
        pltpu.make_async_copy(dst2.at[pl.ds(base128, G)], coh[1],
                              coh[3]).wait()
        plsc.subcore_barrier()

        rpt = NPAD // NS
        pltpu.sync_copy(acc.at[pl.ds(s * rpt, rpt)],
                        out.at[pl.ds(s * rpt, rpt), c])

    return segsum


_SEGSUM_CACHE = {}


def _segsum(xs, src, dst):
    E = src.shape[0]
    nch = -(-E // (NS * CB))
    nch = nch + (nch % 2)
    epad = nch * (NS * CB)
    if nch not in _SEGSUM_CACHE:
        _SEGSUM_CACHE[nch] = _make_segsum(nch)
    pad_s = jnp.full((epad - E,), 0, jnp.int32)
    pad_d = jnp.full((epad - E,), NOUT, jnp.int32)
    s2 = jnp.concatenate([src, pad_s]).reshape(-1, 128)
    d2 = jnp.concatenate([dst, pad_d]).reshape(-1, 128)
    x2 = xs.reshape(-1, DH)
    out = _SEGSUM_CACHE[nch](x2, s2, d2)
    return out[:NOUT].reshape(NOUT, D)


_H0 = E_UU + E_II
_H1 = 2 * E_UI
_H0P = -(-_H0 // (NS * CB)) * (NS * CB)
_H1P = -(-_H1 // (NS * CB)) * (NS * CB)
_H0CH = _H0P // (NS * CB)
_H1CH = _H1P // (NS * CB)


_HIST_CACHE = []


def _make_hist():
    @functools.partial(
        pl.kernel,
        mesh=_mesh(),
        out_type=jax.ShapeDtypeStruct((2 * HPAD, 16), jnp.float32),
        scratch_types=(
            pltpu.VMEM_SHARED((HPAD, 16), jnp.float32),
            pltpu.VMEM((G, 128), jnp.int32),
            pltpu.VMEM((CB, 16), jnp.float32),
            pltpu.VMEM((ZR, 16), jnp.float32),
        ),
        compiler_params=pltpu.CompilerParams(use_tc_tiling_on_sc=False),
    )
    def _hist_kernel(dst2, out, acc, didx, ones, zbuf):
        c = lax.axis_index("c")
        s = lax.axis_index("s")
        _zero_fill(zbuf, ZR, 1)
        for z in range(HPAD // NS // ZR):
            pltpu.sync_copy(zbuf, acc.at[pl.ds(s * (HPAD // NS) + z * ZR, ZR)])

        ovec = jnp.ones((16,), jnp.float32)

        def ob(i, _):
            ones[i, pl.ds(0, 16)] = ovec
            return 0

        lax.fori_loop(0, CB, ob, 0)
        plsc.subcore_barrier()

        ept128_0 = _H0CH * G
        ept128_1 = _H1CH * G
        nch = jnp.where(c == 0, _H0CH, _H1CH)
        base128 = jnp.where(c == 0, s * ept128_0, _H0P // 128 + s * ept128_1)

        def chunk(g, _):
            off = base128 + g * G
            pltpu.sync_copy(dst2.at[pl.ds(off, G)], didx)
            for j in range(G):
                pltpu.sync_copy(ones.at[pl.ds(j * 128, 128)],
                                acc.at[didx.at[j]], add=True)
            return 0

        lax.fori_loop(0, nch, chunk, 0)
        plsc.subcore_barrier()

        rpt = HPAD // NS
        pltpu.sync_copy(acc.at[pl.ds(s * rpt, rpt)],
                        out.at[pl.ds(c * HPAD + s * rpt, rpt)])

    return _hist_kernel


def _hist_call(d):
    if not _HIST_CACHE:
        _HIST_CACHE.append(_make_hist())
    return _HIST_CACHE[0](d)


def _degrees(uu_row, ii_row, ui_row, ui_col):
    p0 = jnp.full((_H0P - _H0,), HN, jnp.int32)
    p1 = jnp.full((_H1P - _H1,), HN, jnp.int32)
    d = jnp.concatenate([
        uu_row, ii_row + U_N, p0,
        ui_row, ui_col + U_N, p1,
    ]).reshape(-1, 128)
    cnt = _hist_call(d)[:, 0]
    return (cnt[:U_N], cnt[U_N:2 * U_N],
            cnt[HPAD:HPAD + U_N], cnt[HPAD + U_N:HPAD + 2 * U_N])


def _l2norm(x):
    n = jnp.sqrt(jnp.sum(x * x, axis=-1, keepdims=True))
    return x / jnp.maximum(n, 1e-12)


def _dis(cnt):
    return jnp.where(cnt > 0, lax.rsqrt(jnp.maximum(cnt, 1e-12)), 0.0)


def _mlp(x, pw, pb, a, ow, ob):
    h = x @ pw + pb
    h = jnp.where(h >= 0, h, a * h)
    h = h @ ow + ob
    return _l2norm(h)


def kernel(user_emb, item_emb, gating_weightu, gating_weightub,
           gating_weighti, gating_weightib, meta_netu_w, meta_netu_b,
           meta_neti_w, meta_neti_b, mlp_pre_w, mlp_pre_b, mlp_prelu,
           mlp_out_w, mlp_out_b, uu_row, uu_col, ii_row, ii_col, ui_row,
           ui_col):
    cnt_uu, cnt_ii, cnt_su, cnt_si = _degrees(uu_row, ii_row, ui_row, ui_col)
    dis_uu = _dis(cnt_uu)[:, None]
    dis_ii = _dis(cnt_ii)[:, None]
    dis_su = _dis(cnt_su)[:, None]
    dis_si = _dis(cnt_si)[:, None]

    uu0 = user_emb * jax.nn.sigmoid(user_emb @ gating_weightu + gating_weightub)
    ii0 = item_emb * jax.nn.sigmoid(item_emb @ gating_weighti + gating_weightib)

    all_u, all_i = [uu0], [ii0]
    all_ui_u, all_ui_i = [user_emb], [item_emb]
    uemb, iemb = uu0, ii0
    ui_u, ui_i = user_emb, item_emb
    for _ in range(LAYERS):
        u0 = dis_uu * _segsum(dis_uu * uemb, uu_col, uu_row)
        i0 = dis_ii * _segsum(dis_ii * iemb, ii_col, ii_row)
        ui_lu = dis_su * _segsum(dis_si * ui_i, ui_col, ui_row)
        ui_li = dis_si * _segsum(dis_su * ui_u, ui_row, ui_col)
        uemb = (u0 + ui_lu) / 2.0
        iemb = (i0 + ui_li) / 2.0
        ui_u, ui_i = uemb, iemb
        all_u.append(_l2norm(u0))
        all_i.append(_l2norm(i0))
        all_ui_u.append(_l2norm(ui_lu))
        all_ui_i.append(_l2norm(ui_li))

    userEmbedding = sum(all_u) / (LAYERS + 1)
    itemEmbedding = sum(all_i) / (LAYERS + 1)
    ui_userEmbedding = sum(all_ui_u) / (LAYERS + 1)
    ui_itemEmbedding = sum(all_ui_i) / (LAYERS + 1)

    uneighbor = _segsum(ui_itemEmbedding, ui_col, ui_row)
    ineighbor = _segsum(ui_userEmbedding, ui_row, ui_col)
    tembedu = jnp.concatenate([userEmbedding, ui_userEmbedding, uneighbor],
                              1) @ meta_netu_w + meta_netu_b
    tembedi = jnp.concatenate([itemEmbedding, ui_itemEmbedding, ineighbor],
                              1) @ meta_neti_w + meta_neti_b
    metau1 = _mlp(tembedu, mlp_pre_w[0], mlp_pre_b[0], mlp_prelu[0],
                  mlp_out_w[0], mlp_out_b[0]).reshape(-1, D, K)
    metau2 = _mlp(tembedu, mlp_pre_w[1], mlp_pre_b[1], mlp_prelu[1],
                  mlp_out_w[1], mlp_out_b[1]).reshape(-1, K, D)
    metai1 = _mlp(tembedi, mlp_pre_w[2], mlp_pre_b[2], mlp_prelu[2],
                  mlp_out_w[2], mlp_out_b[2]).reshape(-1, D, K)
    metai2 = _mlp(tembedi, mlp_pre_w[3], mlp_pre_b[3], mlp_prelu[3],
                  mlp_out_w[3], mlp_out_b[3]).reshape(-1, K, D)
    lw_u1 = jax.nn.softmax(metau1 + jnp.mean(metau1, 0), axis=1)
    lw_u2 = jax.nn.softmax(metau2 + jnp.mean(metau2, 0), axis=1)
    lw_i1 = jax.nn.softmax(metai1 + jnp.mean(metai1, 0), axis=1)
    lw_i2 = jax.nn.softmax(metai2 + jnp.mean(metai2, 0), axis=1)
    tu = jnp.sum(userEmbedding[:, :, None] * lw_u1, axis=1)
    tu = jnp.sum(tu[:, :, None] * lw_u2, axis=1)
    ti = jnp.sum(itemEmbedding[:, :, None] * lw_i1, axis=1)
    ti = jnp.sum(ti[:, :, None] * lw_i2, axis=1)
    userEmbedding = userEmbedding + tu
    itemEmbedding = itemEmbedding + ti
    return userEmbedding, itemEmbedding, ui_userEmbedding, ui_itemEmbedding
